# Initial kernel scaffold; baseline (speedup 1.0000x reference)
import jax
import jax.numpy as jnp
from jax.experimental import pallas as pl


def kernel(input, rois):
    raise NotImplementedError("write your pallas kernel here")


# speedup vs baseline: 2.6001x; 2.6001x over previous
import functools

import jax
import jax.numpy as jnp
from jax import lax
from jax.experimental import pallas as pl
from jax.experimental.pallas import tpu as pltpu
from jax.experimental.pallas import tpu_sc as plsc

N, C, H, W = 2, 128, 200, 200
K = 1000
PH, PW = 7, 7
SPATIAL_SCALE = 0.25

NUM_WORKERS = 32
ROIS_PAD = 1024

_out_struct = jax.ShapeDtypeStruct((K, PH * PW, C), jnp.float32)


def _roi_align_sc(feat_flat, rois_pad):
    mesh = plsc.VectorSubcoreMesh(core_axis_name="c", subcore_axis_name="s")

    @functools.partial(
        pl.kernel,
        out_type=_out_struct,
        mesh=mesh,
        scratch_types=[
            pltpu.VMEM((32, 16), jnp.float32),
            pltpu.VMEM((PW * 16,), jnp.int32),
            pltpu.VMEM((PW * 16, C), jnp.float32),
            pltpu.VMEM((PH * PW, C), jnp.float32),
            pltpu.SemaphoreType.DMA,
        ],
    )
    def body(rois_hbm, feat_hbm, out_hbm, rois_v, idx_v, rows_v, out_v, sem):
        wid = lax.axis_index("s") * 2 + lax.axis_index("c")
        k0 = wid * 32
        nk = jnp.minimum(32, K - k0)
        pltpu.sync_copy(rois_hbm.at[pl.ds(k0, 32)], rois_v)

        lane = lax.iota(jnp.int32, 16)
        sy = (lane >> 3) & 1
        sx = (lane >> 2) & 1
        dy = (lane >> 1) & 1
        dx = lane & 1
        sy_f = sy.astype(jnp.float32)
        sx_f = sx.astype(jnp.float32)
        dy_sel = dy == 1
        dx_sel = dx == 1

        def roi_body(i, _):
            r = rois_v[i, :]
            b = r[0]
            x1 = r[1] * SPATIAL_SCALE
            y1 = r[2] * SPATIAL_SCALE
            x2 = r[3] * SPATIAL_SCALE
            y2 = r[4] * SPATIAL_SCALE
            roi_w = jnp.maximum(x2 - x1, 1.0)
            roi_h = jnp.maximum(y2 - y1, 1.0)
            bin_h = roi_h * (1.0 / PH)
            bin_w = roi_w * (1.0 / PW)
            base = (b * float(H * W)).astype(jnp.int32)

            def ph_body(ph, _):
                ph_f = ph.astype(jnp.float32)
                ys = y1 + bin_h * (ph_f + sy_f * 0.5 + 0.25)
                vy = (ys >= -1.0) & (ys <= float(H))
                yc = jnp.clip(ys, 0.0, float(H - 1))
                y0 = yc.astype(jnp.int32)
                ly = yc - y0.astype(jnp.float32)
                wy = jnp.where(dy_sel, ly, 1.0 - ly)
                yi = jnp.minimum(y0 + dy, H - 1)
                ybase = base + yi * W
                wvecs = []
                for pw in range(PW):
                    xs = x1 + bin_w * (float(pw) + sx_f * 0.5 + 0.25)
                    vx = (xs >= -1.0) & (xs <= float(W))
                    xc = jnp.clip(xs, 0.0, float(W - 1))
                    x0 = xc.astype(jnp.int32)
                    lx = xc - x0.astype(jnp.float32)
                    wx = jnp.where(dx_sel, lx, 1.0 - lx)
                    xi = jnp.minimum(x0 + dx, W - 1)
                    w = jnp.where(vy & vx, wy * wx * 0.25, 0.0)
                    idx_v[pl.ds(pw * 16, 16)] = ybase + xi
                    wvecs.append(w)
                pltpu.async_copy(feat_hbm.at[idx_v], rows_v, sem).wait()
                for pw in range(PW):
                    ws = [wvecs[pw][j] for j in range(16)]
                    for cb in range(C // 16):
                        acc = ws[0] * rows_v[pw * 16, pl.ds(cb * 16, 16)]
                        for j in range(1, 16):
                            acc = acc + ws[j] * rows_v[pw * 16 + j, pl.ds(cb * 16, 16)]
                        out_v[ph * PW + pw, pl.ds(cb * 16, 16)] = acc
                return 0

            lax.fori_loop(0, PH, ph_body, 0)
            pltpu.sync_copy(out_v, out_hbm.at[k0 + i])
            return 0

        lax.fori_loop(0, nk, roi_body, 0)

    return body(rois_pad, feat_flat)


def kernel(input, rois):
    feat_flat = input.transpose(0, 2, 3, 1).reshape(N * H * W, C)
    rois_pad = jnp.zeros((ROIS_PAD, 16), jnp.float32).at[:K, :5].set(rois)
    out = _roi_align_sc(feat_flat, rois_pad)
    return out.reshape(K, PH, PW, C).transpose(0, 3, 1, 2)


# speedup vs baseline: 5.5467x; 2.1333x over previous
import functools

import jax
import jax.numpy as jnp
from jax import lax
from jax.experimental import pallas as pl
from jax.experimental.pallas import tpu as pltpu
from jax.experimental.pallas import tpu_sc as plsc

N, C, H, W = 2, 128, 200, 200
K = 1000
PH, PW = 7, 7
SPATIAL_SCALE = 0.25

ROIS_PAD = 1024
CHUNK = PW * 16

_out_struct = jax.ShapeDtypeStruct((K, PH * PW, C), jnp.float32)


def _roi_align_sc(feat_flat, rois_pad):
    mesh = plsc.VectorSubcoreMesh(core_axis_name="c", subcore_axis_name="s")

    @functools.partial(
        pl.kernel,
        out_type=_out_struct,
        mesh=mesh,
        scratch_types=[
            pltpu.VMEM((32, 16), jnp.float32),
            pltpu.VMEM((CHUNK,), jnp.int32),
            pltpu.VMEM((CHUNK,), jnp.int32),
            pltpu.VMEM((CHUNK,), jnp.float32),
            pltpu.VMEM((CHUNK,), jnp.float32),
            pltpu.VMEM((CHUNK, C), jnp.float32),
            pltpu.VMEM((CHUNK, C), jnp.float32),
            pltpu.VMEM((PH * PW, C), jnp.float32),
            pltpu.SemaphoreType.DMA,
            pltpu.SemaphoreType.DMA,
        ],
    )
    def body(rois_hbm, feat_hbm, out_hbm, rois_v,
             idx_a, idx_b, wgt_a, wgt_b, rows_a, rows_b, out_v,
             sem_a, sem_b):
        idx_ab = (idx_a, idx_b)
        wgt_ab = (wgt_a, wgt_b)
        rows_ab = (rows_a, rows_b)
        sem_ab = (sem_a, sem_b)

        wid = lax.axis_index("s") * 2 + lax.axis_index("c")
        k0 = wid * 32
        nk = jnp.minimum(32, K - k0)
        pltpu.sync_copy(rois_hbm.at[pl.ds(k0, 32)], rois_v)

        lane = lax.iota(jnp.int32, 16)
        sy = (lane >> 3) & 1
        sx = (lane >> 2) & 1
        dy = (lane >> 1) & 1
        dx = lane & 1
        sy_f = sy.astype(jnp.float32)
        sx_f = sx.astype(jnp.float32)
        dy_sel = dy == 1
        dx_sel = dx == 1

        def roi_body(i, _):
            r = rois_v[i, :]
            b = r[0]
            x1 = r[1] * SPATIAL_SCALE
            y1 = r[2] * SPATIAL_SCALE
            x2 = r[3] * SPATIAL_SCALE
            y2 = r[4] * SPATIAL_SCALE
            roi_w = jnp.maximum(x2 - x1, 1.0)
            roi_h = jnp.maximum(y2 - y1, 1.0)
            bin_h = roi_h * (1.0 / PH)
            bin_w = roi_w * (1.0 / PW)
            base = (b * float(H * W)).astype(jnp.int32)

            def fill_chunk(ph):
                p = ph & 1
                ph_f = float(ph)
                ys = y1 + bin_h * (ph_f + sy_f * 0.5 + 0.25)
                vy = (ys >= -1.0) & (ys <= float(H))
                yc = jnp.clip(ys, 0.0, float(H - 1))
                y0 = yc.astype(jnp.int32)
                ly = yc - y0.astype(jnp.float32)
                wy = jnp.where(dy_sel, ly, 1.0 - ly)
                yi = jnp.minimum(y0 + dy, H - 1)
                ybase = base + yi * W
                for pw in range(PW):
                    xs = x1 + bin_w * (float(pw) + sx_f * 0.5 + 0.25)
                    vx = (xs >= -1.0) & (xs <= float(W))
                    xc = jnp.clip(xs, 0.0, float(W - 1))
                    x0 = xc.astype(jnp.int32)
                    lx = xc - x0.astype(jnp.float32)
                    wx = jnp.where(dx_sel, lx, 1.0 - lx)
                    xi = jnp.minimum(x0 + dx, W - 1)
                    w = jnp.where(vy & vx, wy * wx * 0.25, 0.0)
                    idx_ab[p][pl.ds(pw * 16, 16)] = ybase + xi
                    wgt_ab[p][pl.ds(pw * 16, 16)] = w
                return pltpu.async_copy(feat_hbm.at[idx_ab[p]], rows_ab[p], sem_ab[p])

            def drain_chunk(ph, copy):
                p = ph & 1
                copy.wait()
                rows = rows_ab[p]
                wgt = wgt_ab[p]

                def bin_body(pw, _):
                    wv = wgt[pl.ds(pw * 16, 16)]
                    ws = [wv[j] for j in range(16)]
                    rbase = pw * 16
                    for cb in range(C // 16):
                        acc = ws[0] * rows[rbase, pl.ds(cb * 16, 16)]
                        for j in range(1, 16):
                            acc = acc + ws[j] * rows[rbase + j, pl.ds(cb * 16, 16)]
                        out_v[ph * PW + pw, pl.ds(cb * 16, 16)] = acc
                    return 0

                lax.fori_loop(0, PW, bin_body, 0)

            copies = {0: fill_chunk(0)}
            for ph in range(PH):
                if ph + 1 < PH:
                    copies[ph + 1] = fill_chunk(ph + 1)
                drain_chunk(ph, copies[ph])
            pltpu.sync_copy(out_v, out_hbm.at[k0 + i])
            return 0

        lax.fori_loop(0, nk, roi_body, 0)

    return body(rois_pad, feat_flat)


def kernel(input, rois):
    feat_flat = input.transpose(0, 2, 3, 1).reshape(N * H * W, C)
    rois_pad = jnp.zeros((ROIS_PAD, 16), jnp.float32).at[:K, :5].set(rois)
    out = _roi_align_sc(feat_flat, rois_pad)
    return out.reshape(K, PH, PW, C).transpose(0, 3, 1, 2)
